# baseline (speedup 1.0000x reference)
import functools

import jax
import jax.numpy as jnp
from jax.experimental import pallas as pl
from jax.experimental.pallas import tpu as pltpu
from jax.experimental.pallas import tpu_sc as plsc

VOCAB = 1000
EMB = 64
HID = 32
G4 = 4 * HID
NCLS = 10
B = 4096
L = 200

GATHER_WINDOW = 128
MACRO_ROWS = 256
GATHER_DTYPE = jnp.float32


def _project_kernel(emb_ref, wih_ref, bias_ref, p_ref):
    p_ref[...] = (
        jnp.dot(emb_ref[...], wih_ref[...], preferred_element_type=jnp.float32)
        + bias_ref[...]
    ).astype(p_ref.dtype)


def _project_table(emb_table, wih_t, bias, dtype):
    return pl.pallas_call(
        _project_kernel,
        out_shape=jax.ShapeDtypeStruct((VOCAB, G4), dtype),
    )(emb_table, wih_t, bias)


def _sc_gather(table, idx_flat):
    n_idx = idx_flat.shape[0]
    d = table.shape[1]
    mesh = plsc.VectorSubcoreMesh(core_axis_name="c", subcore_axis_name="s")
    n_workers = 32
    per_w = n_idx // n_workers
    w = GATHER_WINDOW
    gpm = MACRO_ROWS // w
    n_macro = per_w // MACRO_ROWS

    @functools.partial(
        pl.kernel,
        out_type=jax.ShapeDtypeStruct((n_idx, d), table.dtype),
        mesh=mesh,
        scratch_types=[
            pltpu.VMEM((per_w,), jnp.int32),
            pltpu.VMEM((MACRO_ROWS, d), table.dtype),
            pltpu.VMEM((MACRO_ROWS, d), table.dtype),
            pltpu.SemaphoreType.DMA,
            pltpu.SemaphoreType.DMA,
        ],
    )
    def gather_kernel(table_hbm, idx_hbm, out_hbm, idx_v, big0, big1,
                      gsem0, gsem1):
        wid = jax.lax.axis_index("s") * 2 + jax.lax.axis_index("c")
        base = wid * per_w
        pltpu.sync_copy(idx_hbm.at[pl.ds(base, per_w)], idx_v)

        def fire_macro(m, big, sem):
            for q in range(gpm):
                pltpu.async_copy(
                    table_hbm.at[idx_v.at[pl.ds(m * MACRO_ROWS + q * w, w)]],
                    big.at[pl.ds(q * w, w)], sem)

        def drain_macro(big, sem):
            for q in range(gpm):
                pltpu.make_async_copy(
                    table_hbm.at[idx_v.at[pl.ds(0, w)]],
                    big.at[pl.ds(q * w, w)], sem).wait()

        fire_macro(0, big0, gsem0)

        @pl.loop(0, n_macro, step=2)
        def _(m):
            fire_macro(m + 1, big1, gsem1)
            drain_macro(big0, gsem0)
            pltpu.sync_copy(
                big0, out_hbm.at[pl.ds(base + m * MACRO_ROWS, MACRO_ROWS)])

            @pl.when(m + 2 < n_macro)
            def _():
                fire_macro(m + 2, big0, gsem0)

            drain_macro(big1, gsem1)
            pltpu.sync_copy(
                big1,
                out_hbm.at[pl.ds(base + (m + 1) * MACRO_ROWS, MACRO_ROWS)])

    return gather_kernel(table, idx_flat)


def _lstm_step_kernel(gx_ref, whh_ref, fcw_ref, fcb_ref,
                      out_ref, h_ref, c_ref):
    t = pl.program_id(0)

    @pl.when(t == 0)
    def _():
        h_ref[...] = jnp.zeros_like(h_ref)
        c_ref[...] = jnp.zeros_like(c_ref)

    h = h_ref[...]
    gates = gx_ref[0].astype(jnp.float32) + jnp.dot(
        h, whh_ref[...], preferred_element_type=jnp.float32)
    i = jax.nn.sigmoid(gates[:, 0 * HID:1 * HID])
    f = jax.nn.sigmoid(gates[:, 1 * HID:2 * HID])
    g = jnp.tanh(gates[:, 2 * HID:3 * HID])
    o = jax.nn.sigmoid(gates[:, 3 * HID:4 * HID])
    c = f * c_ref[...] + i * g
    h = o * jnp.tanh(c)
    c_ref[...] = c
    h_ref[...] = h

    @pl.when(t == L - 1)
    def _():
        out_ref[...] = (
            jnp.dot(h, fcw_ref[...], preferred_element_type=jnp.float32)
            + fcb_ref[...]
        )


def _tc_lstm(gx, whh_t, fcw_t, fcb):
    return pl.pallas_call(
        _lstm_step_kernel,
        grid=(L,),
        in_specs=[
            pl.BlockSpec((1, B, G4), lambda t: (t, 0, 0)),
            pl.BlockSpec((HID, G4), lambda t: (0, 0)),
            pl.BlockSpec((HID, NCLS), lambda t: (0, 0)),
            pl.BlockSpec((1, NCLS), lambda t: (0, 0)),
        ],
        out_specs=pl.BlockSpec((B, NCLS), lambda t: (0, 0)),
        out_shape=jax.ShapeDtypeStruct((B, NCLS), jnp.float32),
        scratch_shapes=[
            pltpu.VMEM((B, HID), jnp.float32),
            pltpu.VMEM((B, HID), jnp.float32),
        ],
    )(gx, whh_t, fcw_t, fcb)


@jax.jit
def kernel(text, emb_table, w_ih, w_hh, b_ih, b_hh, fc_w, fc_b):
    idx_flat = text.T.astype(jnp.int32).reshape(L * B)

    wih_t = w_ih.T
    bias = (b_ih + b_hh).reshape(1, G4)
    proj = _project_table(emb_table, wih_t, bias, GATHER_DTYPE)

    gx = _sc_gather(proj, idx_flat).reshape(L, B, G4)

    whh_t = w_hh.T
    fcw_t = fc_w.T
    fcb = fc_b.reshape(1, NCLS)
    return _tc_lstm(gx, whh_t, fcw_t, fcb)


# speedup vs baseline: 1.7988x; 1.7988x over previous
import functools

import jax
import jax.numpy as jnp
from jax.experimental import pallas as pl
from jax.experimental.pallas import tpu as pltpu
from jax.experimental.pallas import tpu_sc as plsc

VOCAB = 1000
EMB = 64
HID = 32
G4 = 4 * HID
NCLS = 10
B = 4096
L = 200

GATHER_WINDOW = 128
GATHER_DTYPE = jnp.float32
CHUNK_L = 40


def _project_kernel(emb_ref, wih_ref, bias_ref, p_ref):
    p_ref[...] = (
        jnp.dot(emb_ref[...], wih_ref[...], preferred_element_type=jnp.float32)
        + bias_ref[...]
    ).astype(p_ref.dtype)


def _project_table(emb_table, wih_t, bias, dtype):
    return pl.pallas_call(
        _project_kernel,
        out_shape=jax.ShapeDtypeStruct((VOCAB, G4), dtype),
    )(emb_table, wih_t, bias)


def _sc_gather(table, idx_flat):
    n_idx = idx_flat.shape[0]
    d = table.shape[1]
    mesh = plsc.VectorSubcoreMesh(core_axis_name="c", subcore_axis_name="s")
    n_workers = 32
    per_w = n_idx // n_workers
    w = GATHER_WINDOW
    n_chunks = per_w // w
    nbuf = 4
    @functools.partial(
        pl.kernel,
        out_type=jax.ShapeDtypeStruct((n_idx, d), table.dtype),
        mesh=mesh,
        scratch_types=[
            pltpu.VMEM((per_w,), jnp.int32),
            pltpu.VMEM((w, d), table.dtype),
            pltpu.VMEM((w, d), table.dtype),
            pltpu.SemaphoreType.DMA,
            pltpu.SemaphoreType.DMA,
        ],
    )
    def gather_kernel(table_hbm, idx_hbm, out_hbm, idx_v, rows0, rows1,
                      gsem0, gsem1):
        wid = jax.lax.axis_index("s") * 2 + jax.lax.axis_index("c")
        base = wid * per_w
        pltpu.sync_copy(idx_hbm.at[pl.ds(base, per_w)], idx_v)

        def start_gather(chunk, rows, sem):
            pltpu.async_copy(
                table_hbm.at[idx_v.at[pl.ds(chunk * w, w)]], rows, sem)

        def wait_gather(rows, sem):
            pltpu.make_async_copy(
                table_hbm.at[idx_v.at[pl.ds(0, w)]], rows, sem).wait()

        start_gather(0, rows0, gsem0)

        @pl.loop(0, n_chunks, step=2)
        def _(ck):
            start_gather(ck + 1, rows1, gsem1)
            wait_gather(rows0, gsem0)
            pltpu.sync_copy(rows0, out_hbm.at[pl.ds(base + ck * w, w)])

            @pl.when(ck + 2 < n_chunks)
            def _():
                start_gather(ck + 2, rows0, gsem0)

            wait_gather(rows1, gsem1)
            pltpu.sync_copy(rows1, out_hbm.at[pl.ds(base + (ck + 1) * w, w)])

    return gather_kernel(table, idx_flat)


def _lstm_chunk_kernel(gx_ref, whh_ref, fcw_ref, fcb_ref, hin_ref, cin_ref,
                       hout_ref, cout_ref, out_ref):
    t = pl.program_id(0)

    @pl.when(t == 0)
    def _():
        hout_ref[...] = hin_ref[...]
        cout_ref[...] = cin_ref[...]

    h = hout_ref[...]
    gx_t = jnp.transpose(gx_ref[0].astype(jnp.float32))
    gates = gx_t + jnp.dot(
        whh_ref[...], h, preferred_element_type=jnp.float32)

    def sigmoid(x):
        return 0.5 * jnp.tanh(0.5 * x) + 0.5

    i = sigmoid(gates[0 * HID:1 * HID, :])
    f = sigmoid(gates[1 * HID:2 * HID, :])
    g = jnp.tanh(gates[2 * HID:3 * HID, :])
    o = sigmoid(gates[3 * HID:4 * HID, :])
    c = f * cout_ref[...] + i * g
    h = o * jnp.tanh(c)
    cout_ref[...] = c
    hout_ref[...] = h

    @pl.when(t == CHUNK_L - 1)
    def _():
        out_t = jnp.dot(
            fcw_ref[...], h, preferred_element_type=jnp.float32)
        out_ref[...] = jnp.transpose(out_t) + fcb_ref[...]


def _tc_lstm_chunk(gx, whh, fcw, fcb, h, c):
    return pl.pallas_call(
        _lstm_chunk_kernel,
        grid=(CHUNK_L,),
        in_specs=[
            pl.BlockSpec((1, B, G4), lambda t: (t, 0, 0)),
            pl.BlockSpec((G4, HID), lambda t: (0, 0)),
            pl.BlockSpec((NCLS, HID), lambda t: (0, 0)),
            pl.BlockSpec((1, NCLS), lambda t: (0, 0)),
            pl.BlockSpec((HID, B), lambda t: (0, 0)),
            pl.BlockSpec((HID, B), lambda t: (0, 0)),
        ],
        out_specs=[
            pl.BlockSpec((HID, B), lambda t: (0, 0)),
            pl.BlockSpec((HID, B), lambda t: (0, 0)),
            pl.BlockSpec((B, NCLS), lambda t: (0, 0)),
        ],
        out_shape=[
            jax.ShapeDtypeStruct((HID, B), jnp.float32),
            jax.ShapeDtypeStruct((HID, B), jnp.float32),
            jax.ShapeDtypeStruct((B, NCLS), jnp.float32),
        ],
        input_output_aliases={4: 0, 5: 1},
    )(gx, whh, fcw, fcb, h, c)


@jax.jit
def kernel(text, emb_table, w_ih, w_hh, b_ih, b_hh, fc_w, fc_b):
    idx_flat = text.T.astype(jnp.int32).reshape(L * B)

    wih_t = w_ih.T
    bias = (b_ih + b_hh).reshape(1, G4)
    proj = _project_table(emb_table, wih_t, bias, GATHER_DTYPE)

    fcb = fc_b.reshape(1, NCLS)
    h = jnp.zeros((HID, B), jnp.float32)
    c = jnp.zeros((HID, B), jnp.float32)
    out = None
    for k in range(L // CHUNK_L):
        idx_k = jax.lax.dynamic_slice_in_dim(
            idx_flat, k * CHUNK_L * B, CHUNK_L * B)
        gx_k = _sc_gather(proj, idx_k).reshape(CHUNK_L, B, G4)
        h, c, out = _tc_lstm_chunk(gx_k, w_hh, fc_w, fcb, h, c)
    return out
